# baseline (speedup 1.0000x reference)
import functools

import jax
import jax.numpy as jnp
from jax import lax
from jax.experimental import pallas as pl
from jax.experimental.pallas import tpu as pltpu
from jax.experimental.pallas import tpu_sc as plsc

N = 100000
E = 6400000
SUB = 128
CE = 640
NSUB = CE // SUB
LANES = 16
GPS = SUB // LANES
NW = 32
NCHUNK = E // CE
BASE_CHUNKS = NCHUNK // NW
EXTRA = NCHUNK - BASE_CHUNKS * NW


def _rsqrt(x):
    i = lax.bitcast_convert_type(x, jnp.int32)
    i = jnp.int32(0x5F3759DF) - lax.shift_right_logical(i, 1)
    y = lax.bitcast_convert_type(i, jnp.float32)
    for _ in range(3):
        y = y * (jnp.float32(1.5) - jnp.float32(0.5) * x * y * y)
    return y


_MESH = plsc.VectorSubcoreMesh(core_axis_name="c", subcore_axis_name="s")


@functools.partial(
    pl.kernel,
    out_type=[
        jax.ShapeDtypeStruct((2, N, 4), jnp.float32),
        jax.ShapeDtypeStruct((E,), jnp.float32),
    ],
    mesh=_MESH,
    compiler_params=pltpu.CompilerParams(
        needs_layout_passes=False, use_tc_tiling_on_sc=False),
    scratch_types=[
        pltpu.VMEM((NSUB, SUB), jnp.int32),
        pltpu.VMEM((NSUB, SUB), jnp.int32),
        pltpu.VMEM((NSUB, SUB, 8), jnp.float32),
        pltpu.VMEM((NSUB, SUB, 8), jnp.float32),
        pltpu.VMEM((CE,), jnp.float32),
        pltpu.VMEM((CE,), jnp.float32),
        pltpu.VMEM((CE,), jnp.float32),
        pltpu.VMEM((NSUB, SUB, 4), jnp.float32),
        pltpu.VMEM((NSUB, SUB, 4), jnp.float32),
        pltpu.VMEM_SHARED((N, 4), jnp.float32),
        pltpu.SemaphoreType.DMA,
        pltpu.SemaphoreType.DMA,
    ],
)
def _sc_solve(tab, ii, jj, l_in, d0_in, zeros_hbm,
              parts, l_out,
              idx_i, idx_j, rows_i, rows_j, l_v, d0_v, lout_v, ci_v, cj_v,
              acc, sl, ss):
    c = lax.axis_index("c")
    s = lax.axis_index("s")
    w = s * 2 + c

    @pl.when(s == 0)
    def _():
        pltpu.sync_copy(zeros_hbm, acc)

    plsc.subcore_barrier()

    ibase = lax.iota(jnp.int32, LANES)
    col = [jnp.full((LANES,), k, jnp.int32) for k in range(5)]
    ccol = [jnp.full((LANES,), k, jnp.int32) for k in range(3)]

    nch = jnp.where(w < EXTRA, BASE_CHUNKS + 1, BASE_CHUNKS)

    def body(t, _):
        chunk = t * NW + w
        ebase = chunk * CE
        gbase = chunk * NSUB

        pltpu.sync_copy(ii.at[pl.ds(gbase, NSUB)], idx_i)
        pltpu.sync_copy(jj.at[pl.ds(gbase, NSUB)], idx_j)
        pltpu.async_copy(l_in.at[pl.ds(ebase, CE)], l_v, sl)
        pltpu.async_copy(d0_in.at[pl.ds(ebase, CE)], d0_v, sl)
        for g in range(NSUB):
            pltpu.async_copy(tab.at[idx_i.at[g]], rows_i.at[g], sl)
            pltpu.async_copy(tab.at[idx_j.at[g]], rows_j.at[g], sl)
        pltpu.make_async_copy(l_in.at[pl.ds(ebase, CE)], l_v, sl).wait()
        pltpu.make_async_copy(d0_in.at[pl.ds(ebase, CE)], d0_v, sl).wait()
        for g in range(NSUB):
            pltpu.make_async_copy(tab.at[idx_i.at[g]], rows_i.at[g], sl).wait()
            pltpu.make_async_copy(tab.at[idx_j.at[g]], rows_j.at[g], sl).wait()

        def group_body(g16, _):
            sub = g16 // GPS
            rowv = ibase + (g16 % GPS) * LANES
            gv = jnp.full((LANES,), 0, jnp.int32) + sub
            xi = plsc.load_gather(rows_i, [gv, rowv, col[0]])
            yi = plsc.load_gather(rows_i, [gv, rowv, col[1]])
            zi = plsc.load_gather(rows_i, [gv, rowv, col[2]])
            wi = plsc.load_gather(rows_i, [gv, rowv, col[3]])
            ki = plsc.load_gather(rows_i, [gv, rowv, col[4]])
            xj = plsc.load_gather(rows_j, [gv, rowv, col[0]])
            yj = plsc.load_gather(rows_j, [gv, rowv, col[1]])
            zj = plsc.load_gather(rows_j, [gv, rowv, col[2]])
            wj = plsc.load_gather(rows_j, [gv, rowv, col[3]])
            kj = plsc.load_gather(rows_j, [gv, rowv, col[4]])
            dx = xi - xj
            dy = yi - yj
            dz = zi - zj
            dsq = dx * dx + dy * dy + dz * dz
            rinv = _rsqrt(dsq)
            dist = dsq * rinv
            lv = l_v[pl.ds(g16 * LANES, LANES)]
            d0v = d0_v[pl.ds(g16 * LANES, LANES)]
            cons = dist - d0v
            a = (ki + kj) * jnp.float32(0.5)
            ssum = wi + wj
            ssum = jnp.where(ssum == jnp.float32(0.0), jnp.float32(jnp.inf),
                             ssum)
            ldel = (-cons - a * lv) / (ssum + a)
            lout_v[pl.ds(g16 * LANES, LANES)] = lv + ldel
            rn = jnp.where(dsq > jnp.float32(0.0), rinv, jnp.float32(jnp.inf))
            fi = wi * ldel * rn
            fj = -(wj * ldel * rn)
            plsc.store_scatter(ci_v, [gv, rowv, ccol[0]], fi * dx)
            plsc.store_scatter(ci_v, [gv, rowv, ccol[1]], fi * dy)
            plsc.store_scatter(ci_v, [gv, rowv, ccol[2]], fi * dz)
            plsc.store_scatter(cj_v, [gv, rowv, ccol[0]], fj * dx)
            plsc.store_scatter(cj_v, [gv, rowv, ccol[1]], fj * dy)
            plsc.store_scatter(cj_v, [gv, rowv, ccol[2]], fj * dz)
            return 0

        lax.fori_loop(0, NSUB * GPS, group_body, 0)

        pltpu.async_copy(lout_v, l_out.at[pl.ds(ebase, CE)], ss)
        for g in range(NSUB):
            pltpu.sync_copy(ci_v.at[g], acc.at[idx_i.at[g]], add=True)
            pltpu.sync_copy(cj_v.at[g], acc.at[idx_j.at[g]], add=True)
        pltpu.make_async_copy(lout_v, l_out.at[pl.ds(ebase, CE)], ss).wait()
        return 0

    lax.fori_loop(0, nch, body, 0)

    plsc.subcore_barrier()

    @pl.when(s == 0)
    def _():
        pltpu.sync_copy(acc, parts.at[c])


def _combine_body(vp_ref, p0_ref, p1_ref, o_ref):
    o_ref[...] = vp_ref[...] + p0_ref[...] + p1_ref[...]


def _combine(v_pad, parts):
    vp = v_pad.reshape(N * 4 // 128, 128)
    p0 = parts[0].reshape(N * 4 // 128, 128)
    p1 = parts[1].reshape(N * 4 // 128, 128)
    out = pl.pallas_call(
        _combine_body,
        out_shape=jax.ShapeDtypeStruct((N * 4 // 128, 128), jnp.float32),
    )(vp, p0, p1)
    return out.reshape(N, 4)[:, :3]


def kernel(V_predict, L, V_w, V_compliance, C_dist, C_init_d):
    tab = jnp.concatenate(
        [V_predict, V_w, V_compliance, jnp.zeros((N, 3), jnp.float32)], axis=1)
    ii = C_dist[:, 0].reshape(E // SUB, SUB)
    jj = C_dist[:, 1].reshape(E // SUB, SUB)
    zeros_hbm = jnp.zeros((N, 4), jnp.float32)
    parts, l_new = _sc_solve(tab, ii, jj, L.reshape(E), C_init_d.reshape(E),
                             zeros_hbm)
    v_pad = jnp.concatenate([V_predict, jnp.zeros((N, 1), jnp.float32)], axis=1)
    v_out = _combine(v_pad, parts)
    return (v_out, l_new.reshape(E, 1))


# speedup vs baseline: 1.3643x; 1.3643x over previous
import functools

import jax
import jax.numpy as jnp
from jax import lax
from jax.experimental import pallas as pl
from jax.experimental.pallas import tpu as pltpu
from jax.experimental.pallas import tpu_sc as plsc

N = 100000
E = 6400000
SUB = 128
CE = 640
NSUB = CE // SUB
LANES = 16
GPS = SUB // LANES
NW = 32
NCHUNK = E // CE
BASE_CHUNKS = (NCHUNK // NW) & ~1
EXTRA = NCHUNK - BASE_CHUNKS * NW
NP = BASE_CHUNKS // 2


def _rsqrt(x):
    i = lax.bitcast_convert_type(x, jnp.int32)
    i = jnp.int32(0x5F3759DF) - lax.shift_right_logical(i, 1)
    y = lax.bitcast_convert_type(i, jnp.float32)
    for _ in range(3):
        y = y * (jnp.float32(1.5) - jnp.float32(0.5) * x * y * y)
    return y


_MESH = plsc.VectorSubcoreMesh(core_axis_name="c", subcore_axis_name="s")

_BUF = [
    pltpu.VMEM((NSUB, SUB), jnp.int32),
    pltpu.VMEM((NSUB, SUB), jnp.int32),
    pltpu.VMEM((NSUB, SUB, 8), jnp.float32),
    pltpu.VMEM((NSUB, SUB, 8), jnp.float32),
    pltpu.VMEM((CE,), jnp.float32),
    pltpu.VMEM((CE,), jnp.float32),
    pltpu.VMEM((CE,), jnp.float32),
    pltpu.VMEM((NSUB, SUB, 4), jnp.float32),
    pltpu.VMEM((NSUB, SUB, 4), jnp.float32),
    pltpu.SemaphoreType.DMA,
    pltpu.SemaphoreType.DMA,
]


@functools.partial(
    pl.kernel,
    out_type=[
        jax.ShapeDtypeStruct((2, N, 4), jnp.float32),
        jax.ShapeDtypeStruct((E,), jnp.float32),
    ],
    mesh=_MESH,
    compiler_params=pltpu.CompilerParams(
        needs_layout_passes=False, use_tc_tiling_on_sc=False),
    scratch_types=_BUF + _BUF + [
        pltpu.VMEM_SHARED((N, 4), jnp.float32),
    ],
)
def _sc_solve(tab, ii, jj, l_in, d0_in, zeros_hbm,
              parts, l_out,
              ia_i, ia_j, ra_i, ra_j, la, d0a, loa, cia, cja, sla, ssa,
              ib_i, ib_j, rb_i, rb_j, lb, d0b, lob, cib, cjb, slb, ssb,
              acc):
    c = lax.axis_index("c")
    s = lax.axis_index("s")
    w = s * 2 + c

    @pl.when(s == 0)
    def _():
        pltpu.sync_copy(zeros_hbm, acc)

    plsc.subcore_barrier()

    A = (ia_i, ia_j, ra_i, ra_j, la, d0a, loa, cia, cja, sla, ssa)
    B = (ib_i, ib_j, rb_i, rb_j, lb, d0b, lob, cib, cjb, slb, ssb)

    ibase = lax.iota(jnp.int32, LANES)
    col = [jnp.full((LANES,), k, jnp.int32) for k in range(5)]
    ccol = [jnp.full((LANES,), k, jnp.int32) for k in range(3)]

    def load_start(chunk, bufs):
        idx_i, idx_j, rows_i, rows_j, l_v, d0_v = bufs[:6]
        sl = bufs[9]
        ebase = chunk * CE
        gbase = chunk * NSUB
        pltpu.sync_copy(ii.at[pl.ds(gbase, NSUB)], idx_i)
        pltpu.sync_copy(jj.at[pl.ds(gbase, NSUB)], idx_j)
        pltpu.async_copy(l_in.at[pl.ds(ebase, CE)], l_v, sl)
        pltpu.async_copy(d0_in.at[pl.ds(ebase, CE)], d0_v, sl)
        for g in range(NSUB):
            pltpu.async_copy(tab.at[idx_i.at[g]], rows_i.at[g], sl)
            pltpu.async_copy(tab.at[idx_j.at[g]], rows_j.at[g], sl)

    def load_wait(chunk, bufs):
        idx_i, idx_j, rows_i, rows_j, l_v, d0_v = bufs[:6]
        sl = bufs[9]
        ebase = chunk * CE
        pltpu.make_async_copy(l_in.at[pl.ds(ebase, CE)], l_v, sl).wait()
        pltpu.make_async_copy(d0_in.at[pl.ds(ebase, CE)], d0_v, sl).wait()
        for g in range(NSUB):
            pltpu.make_async_copy(tab.at[idx_i.at[g]], rows_i.at[g], sl).wait()
            pltpu.make_async_copy(tab.at[idx_j.at[g]], rows_j.at[g], sl).wait()

    def compute(bufs):
        _, _, rows_i, rows_j, l_v, d0_v, lout_v, ci_v, cj_v = bufs[:9]

        def group_body(g16, _):
            sub = g16 // GPS
            rowv = ibase + (g16 % GPS) * LANES
            gv = jnp.full((LANES,), 0, jnp.int32) + sub
            xi = plsc.load_gather(rows_i, [gv, rowv, col[0]])
            yi = plsc.load_gather(rows_i, [gv, rowv, col[1]])
            zi = plsc.load_gather(rows_i, [gv, rowv, col[2]])
            wi = plsc.load_gather(rows_i, [gv, rowv, col[3]])
            ki = plsc.load_gather(rows_i, [gv, rowv, col[4]])
            xj = plsc.load_gather(rows_j, [gv, rowv, col[0]])
            yj = plsc.load_gather(rows_j, [gv, rowv, col[1]])
            zj = plsc.load_gather(rows_j, [gv, rowv, col[2]])
            wj = plsc.load_gather(rows_j, [gv, rowv, col[3]])
            kj = plsc.load_gather(rows_j, [gv, rowv, col[4]])
            dx = xi - xj
            dy = yi - yj
            dz = zi - zj
            dsq = dx * dx + dy * dy + dz * dz
            rinv = _rsqrt(dsq)
            dist = dsq * rinv
            lv = l_v[pl.ds(g16 * LANES, LANES)]
            d0v = d0_v[pl.ds(g16 * LANES, LANES)]
            cons = dist - d0v
            a = (ki + kj) * jnp.float32(0.5)
            ssum = wi + wj
            ssum = jnp.where(ssum == jnp.float32(0.0), jnp.float32(jnp.inf),
                             ssum)
            ldel = (-cons - a * lv) / (ssum + a)
            lout_v[pl.ds(g16 * LANES, LANES)] = lv + ldel
            rn = jnp.where(dsq > jnp.float32(0.0), rinv, jnp.float32(jnp.inf))
            fi = wi * ldel * rn
            fj = -(wj * ldel * rn)
            plsc.store_scatter(ci_v, [gv, rowv, ccol[0]], fi * dx)
            plsc.store_scatter(ci_v, [gv, rowv, ccol[1]], fi * dy)
            plsc.store_scatter(ci_v, [gv, rowv, ccol[2]], fi * dz)
            plsc.store_scatter(cj_v, [gv, rowv, ccol[0]], fj * dx)
            plsc.store_scatter(cj_v, [gv, rowv, ccol[1]], fj * dy)
            plsc.store_scatter(cj_v, [gv, rowv, ccol[2]], fj * dz)
            return 0

        lax.fori_loop(0, NSUB * GPS, group_body, 0)

    def store(chunk, bufs):
        idx_i, idx_j = bufs[:2]
        lout_v, ci_v, cj_v = bufs[6:9]
        ss = bufs[10]
        ebase = chunk * CE
        pltpu.async_copy(lout_v, l_out.at[pl.ds(ebase, CE)], ss)
        for g in range(NSUB):
            pltpu.sync_copy(ci_v.at[g], acc.at[idx_i.at[g]], add=True)
            pltpu.sync_copy(cj_v.at[g], acc.at[idx_j.at[g]], add=True)

    def lout_wait(chunk, bufs):
        lout_v = bufs[6]
        ss = bufs[10]
        ebase = chunk * CE
        pltpu.make_async_copy(lout_v, l_out.at[pl.ds(ebase, CE)], ss).wait()

    def wchunk(t):
        return t * NW + w

    load_start(wchunk(0), A)

    def body(p, _):
        c0 = 2 * p
        c1 = 2 * p + 1

        load_start(wchunk(c1), B)
        load_wait(wchunk(c0), A)

        @pl.when(p >= 1)
        def _():
            lout_wait(wchunk(c0 - 2), A)

        compute(A)
        store(wchunk(c0), A)

        @pl.when(p < NP - 1)
        def _():
            load_start(wchunk(c0 + 2), A)

        load_wait(wchunk(c1), B)

        @pl.when(p >= 1)
        def _():
            lout_wait(wchunk(c1 - 2), B)

        compute(B)
        store(wchunk(c1), B)
        return 0

    lax.fori_loop(0, NP, body, 0)

    lout_wait(wchunk(BASE_CHUNKS - 2), A)
    lout_wait(wchunk(BASE_CHUNKS - 1), B)

    @pl.when(w < EXTRA)
    def _():
        extra_chunk = BASE_CHUNKS * NW + w
        load_start(extra_chunk, A)
        load_wait(extra_chunk, A)
        compute(A)
        store(extra_chunk, A)
        lout_wait(extra_chunk, A)

    plsc.subcore_barrier()

    @pl.when(s == 0)
    def _():
        pltpu.sync_copy(acc, parts.at[c])


def _combine_body(vp_ref, p0_ref, p1_ref, o_ref):
    o_ref[...] = vp_ref[...] + p0_ref[...] + p1_ref[...]


def _combine(v_pad, parts):
    vp = v_pad.reshape(N * 4 // 128, 128)
    p0 = parts[0].reshape(N * 4 // 128, 128)
    p1 = parts[1].reshape(N * 4 // 128, 128)
    out = pl.pallas_call(
        _combine_body,
        out_shape=jax.ShapeDtypeStruct((N * 4 // 128, 128), jnp.float32),
    )(vp, p0, p1)
    return out.reshape(N, 4)[:, :3]


def kernel(V_predict, L, V_w, V_compliance, C_dist, C_init_d):
    tab = jnp.concatenate(
        [V_predict, V_w, V_compliance, jnp.zeros((N, 3), jnp.float32)], axis=1)
    ii = C_dist[:, 0].reshape(E // SUB, SUB)
    jj = C_dist[:, 1].reshape(E // SUB, SUB)
    zeros_hbm = jnp.zeros((N, 4), jnp.float32)
    parts, l_new = _sc_solve(tab, ii, jj, L.reshape(E), C_init_d.reshape(E),
                             zeros_hbm)
    v_pad = jnp.concatenate([V_predict, jnp.zeros((N, 1), jnp.float32)], axis=1)
    v_out = _combine(v_pad, parts)
    return (v_out, l_new.reshape(E, 1))


# speedup vs baseline: 1.6581x; 1.2153x over previous
import functools

import jax
import jax.numpy as jnp
from jax import lax
from jax.experimental import pallas as pl
from jax.experimental.pallas import tpu as pltpu
from jax.experimental.pallas import tpu_sc as plsc

N = 100000
E = 6400000
SUB = 128
CE = 640
NSUB = CE // SUB
LANES = 16
GPS = SUB // LANES
NW = 32
NCHUNK = E // CE
BASE_CHUNKS = (NCHUNK // NW) & ~1
EXTRA = NCHUNK - BASE_CHUNKS * NW
NP = BASE_CHUNKS // 2


def _rsqrt(x):
    i = lax.bitcast_convert_type(x, jnp.int32)
    i = jnp.int32(0x5F3759DF) - lax.shift_right_logical(i, 1)
    y = lax.bitcast_convert_type(i, jnp.float32)
    for _ in range(3):
        y = y * (jnp.float32(1.5) - jnp.float32(0.5) * x * y * y)
    return y


_MESH = plsc.VectorSubcoreMesh(core_axis_name="c", subcore_axis_name="s")

_BUF = [
    pltpu.VMEM((2 * NSUB, SUB), jnp.int32),
    pltpu.VMEM((NSUB, SUB, 8), jnp.float32),
    pltpu.VMEM((NSUB, SUB, 8), jnp.float32),
    pltpu.VMEM((CE,), jnp.float32),
    pltpu.VMEM((CE,), jnp.float32),
    pltpu.VMEM((CE,), jnp.float32),
    pltpu.VMEM((NSUB, SUB, 4), jnp.float32),
    pltpu.VMEM((NSUB, SUB, 4), jnp.float32),
    pltpu.SemaphoreType.DMA,
    pltpu.SemaphoreType.DMA,
]


@functools.partial(
    pl.kernel,
    out_type=[
        jax.ShapeDtypeStruct((2, N, 4), jnp.float32),
        jax.ShapeDtypeStruct((E,), jnp.float32),
    ],
    mesh=_MESH,
    compiler_params=pltpu.CompilerParams(
        needs_layout_passes=False, use_tc_tiling_on_sc=False),
    scratch_types=_BUF + _BUF + [
        pltpu.VMEM_SHARED((N, 4), jnp.float32),
    ],
)
def _sc_solve(tab, idx_hbm, l_in, d0_in, zeros_hbm,
              parts, l_out,
              ia, ra_i, ra_j, la, d0a, loa, cia, cja, sla, ssa,
              ib, rb_i, rb_j, lb, d0b, lob, cib, cjb, slb, ssb,
              acc):
    c = lax.axis_index("c")
    s = lax.axis_index("s")
    w = s * 2 + c

    @pl.when(s == 0)
    def _():
        pltpu.sync_copy(zeros_hbm, acc)

    plsc.subcore_barrier()

    A = (ia, ra_i, ra_j, la, d0a, loa, cia, cja, sla, ssa)
    B = (ib, rb_i, rb_j, lb, d0b, lob, cib, cjb, slb, ssb)

    ibase = lax.iota(jnp.int32, LANES)
    col = [jnp.full((LANES,), k, jnp.int32) for k in range(5)]
    ccol = [jnp.full((LANES,), k, jnp.int32) for k in range(3)]

    def load_start(chunk, bufs):
        idx, rows_i, rows_j, l_v, d0_v = bufs[:5]
        sl = bufs[8]
        ebase = chunk * CE
        gbase = chunk * 2 * NSUB
        pltpu.sync_copy(idx_hbm.at[pl.ds(gbase, 2 * NSUB)], idx)
        pltpu.async_copy(l_in.at[pl.ds(ebase, CE)], l_v, sl)
        pltpu.async_copy(d0_in.at[pl.ds(ebase, CE)], d0_v, sl)
        for g in range(NSUB):
            pltpu.async_copy(tab.at[idx.at[g]], rows_i.at[g], sl)
            pltpu.async_copy(tab.at[idx.at[NSUB + g]], rows_j.at[g], sl)

    def load_wait(chunk, bufs):
        idx, rows_i, rows_j, l_v, d0_v = bufs[:5]
        sl = bufs[8]
        ebase = chunk * CE
        pltpu.make_async_copy(l_in.at[pl.ds(ebase, CE)], l_v, sl).wait()
        pltpu.make_async_copy(d0_in.at[pl.ds(ebase, CE)], d0_v, sl).wait()
        for g in range(NSUB):
            pltpu.make_async_copy(tab.at[idx.at[g]], rows_i.at[g], sl).wait()
            pltpu.make_async_copy(
                tab.at[idx.at[NSUB + g]], rows_j.at[g], sl).wait()

    def compute(bufs):
        _, rows_i, rows_j, l_v, d0_v, lout_v, ci_v, cj_v = bufs[:8]

        @plsc.parallel_loop(0, NSUB * GPS, unroll=2)
        def group_body(g16):
            sub = g16 // GPS
            rowv = ibase + (g16 % GPS) * LANES
            gv = jnp.full((LANES,), 0, jnp.int32) + sub
            xi = plsc.load_gather(rows_i, [gv, rowv, col[0]])
            yi = plsc.load_gather(rows_i, [gv, rowv, col[1]])
            zi = plsc.load_gather(rows_i, [gv, rowv, col[2]])
            wi = plsc.load_gather(rows_i, [gv, rowv, col[3]])
            ki = plsc.load_gather(rows_i, [gv, rowv, col[4]])
            xj = plsc.load_gather(rows_j, [gv, rowv, col[0]])
            yj = plsc.load_gather(rows_j, [gv, rowv, col[1]])
            zj = plsc.load_gather(rows_j, [gv, rowv, col[2]])
            wj = plsc.load_gather(rows_j, [gv, rowv, col[3]])
            kj = plsc.load_gather(rows_j, [gv, rowv, col[4]])
            dx = xi - xj
            dy = yi - yj
            dz = zi - zj
            dsq = dx * dx + dy * dy + dz * dz
            rinv = _rsqrt(dsq)
            dist = dsq * rinv
            lv = l_v[pl.ds(g16 * LANES, LANES)]
            d0v = d0_v[pl.ds(g16 * LANES, LANES)]
            cons = dist - d0v
            a = (ki + kj) * jnp.float32(0.5)
            ssum = wi + wj
            ssum = jnp.where(ssum == jnp.float32(0.0), jnp.float32(jnp.inf),
                             ssum)
            ldel = (-cons - a * lv) / (ssum + a)
            lout_v[pl.ds(g16 * LANES, LANES)] = lv + ldel
            rn = jnp.where(dsq > jnp.float32(0.0), rinv, jnp.float32(jnp.inf))
            fi = wi * ldel * rn
            fj = -(wj * ldel * rn)
            plsc.store_scatter(ci_v, [gv, rowv, ccol[0]], fi * dx)
            plsc.store_scatter(ci_v, [gv, rowv, ccol[1]], fi * dy)
            plsc.store_scatter(ci_v, [gv, rowv, ccol[2]], fi * dz)
            plsc.store_scatter(cj_v, [gv, rowv, ccol[0]], fj * dx)
            plsc.store_scatter(cj_v, [gv, rowv, ccol[1]], fj * dy)
            plsc.store_scatter(cj_v, [gv, rowv, ccol[2]], fj * dz)

    def store(chunk, bufs):
        idx = bufs[0]
        lout_v, ci_v, cj_v = bufs[5:8]
        ss = bufs[9]
        ebase = chunk * CE
        pltpu.async_copy(lout_v, l_out.at[pl.ds(ebase, CE)], ss)
        for g in range(NSUB):
            pltpu.sync_copy(ci_v.at[g], acc.at[idx.at[g]], add=True)
            pltpu.sync_copy(cj_v.at[g], acc.at[idx.at[NSUB + g]], add=True)

    def lout_wait(chunk, bufs):
        lout_v = bufs[5]
        ss = bufs[9]
        ebase = chunk * CE
        pltpu.make_async_copy(lout_v, l_out.at[pl.ds(ebase, CE)], ss).wait()

    def wchunk(t):
        return t * NW + w

    load_start(wchunk(0), A)

    def body(p, _):
        c0 = 2 * p
        c1 = 2 * p + 1

        load_start(wchunk(c1), B)
        load_wait(wchunk(c0), A)

        @pl.when(p >= 1)
        def _():
            lout_wait(wchunk(c0 - 2), A)

        compute(A)
        store(wchunk(c0), A)

        @pl.when(p < NP - 1)
        def _():
            load_start(wchunk(c0 + 2), A)

        load_wait(wchunk(c1), B)

        @pl.when(p >= 1)
        def _():
            lout_wait(wchunk(c1 - 2), B)

        compute(B)
        store(wchunk(c1), B)
        return 0

    lax.fori_loop(0, NP, body, 0)

    lout_wait(wchunk(BASE_CHUNKS - 2), A)
    lout_wait(wchunk(BASE_CHUNKS - 1), B)

    @pl.when(w < EXTRA)
    def _():
        extra_chunk = BASE_CHUNKS * NW + w
        load_start(extra_chunk, A)
        load_wait(extra_chunk, A)
        compute(A)
        store(extra_chunk, A)
        lout_wait(extra_chunk, A)

    plsc.subcore_barrier()

    @pl.when(s == 0)
    def _():
        pltpu.sync_copy(acc, parts.at[c])


def _combine_body(vp_ref, p0_ref, p1_ref, o_ref):
    o_ref[...] = vp_ref[...] + p0_ref[...] + p1_ref[...]


def _combine(v_pad, parts):
    vp = v_pad.reshape(N * 4 // 128, 128)
    p0 = parts[0].reshape(N * 4 // 128, 128)
    p1 = parts[1].reshape(N * 4 // 128, 128)
    out = pl.pallas_call(
        _combine_body,
        out_shape=jax.ShapeDtypeStruct((N * 4 // 128, 128), jnp.float32),
    )(vp, p0, p1)
    return out.reshape(N, 4)[:, :3]


def kernel(V_predict, L, V_w, V_compliance, C_dist, C_init_d):
    tab = jnp.concatenate(
        [V_predict, V_w, V_compliance, jnp.zeros((N, 3), jnp.float32)], axis=1)
    idx_hbm = (C_dist.reshape(NCHUNK, NSUB, SUB, 2)
               .transpose(0, 3, 1, 2)
               .reshape(NCHUNK * 2 * NSUB, SUB))
    zeros_hbm = jnp.zeros((N, 4), jnp.float32)
    parts, l_new = _sc_solve(tab, idx_hbm, L.reshape(E), C_init_d.reshape(E),
                             zeros_hbm)
    v_pad = jnp.concatenate([V_predict, jnp.zeros((N, 1), jnp.float32)], axis=1)
    v_out = _combine(v_pad, parts)
    return (v_out, l_new.reshape(E, 1))
